# baseline (speedup 1.0000x reference)
import functools

import jax
import jax.numpy as jnp
from jax import lax
from jax.experimental import pallas as pl
from jax.experimental.pallas import tpu as pltpu
from jax.experimental.pallas import tpu_sc as plsc

VOCAB_SIZE = 1000000
H = 64
B = 1024
L = 200
TOK = B * L
G = 128
IDX_ROWS = TOK // G
SCALE = float(H) ** 0.5

_info = plsc.get_sparse_core_info()
NC = _info.num_cores
NS = _info.num_subcores
NW = NC * NS
GROUPS_PER_W = TOK // (G * NW)


def _body(table_hbm, idx_hbm, out_hbm, idx_v, rows_v, gsem):
    wid = lax.axis_index("s") * NC + lax.axis_index("c")
    tok0 = wid * GROUPS_PER_W * G
    pltpu.sync_copy(idx_hbm.at[pl.ds(tok0, GROUPS_PER_W * G)], idx_v)

    def group(g, carry):
        pltpu.async_copy(
            table_hbm.at[idx_v.at[pl.ds(g * G, G)]], rows_v, gsem
        ).wait()

        def row16(k, c):
            ivec = idx_v[pl.ds(g * G + k * 16, 16)]
            svec = jnp.where(ivec == jnp.int32(0), jnp.float32(0.0),
                             jnp.float32(SCALE))
            for r in range(16):
                i = k * 16 + r
                s = svec[r]
                for j in range(H // 16):
                    v = rows_v[i, pl.ds(j * 16, 16)]
                    rows_v[i, pl.ds(j * 16, 16)] = v * s
            return c

        lax.fori_loop(0, G // 16, row16, 0)

        pltpu.sync_copy(rows_v, out_hbm.at[pl.ds(tok0 + g * G, G)])
        return carry

    lax.fori_loop(0, GROUPS_PER_W, group, 0)


@jax.jit
def kernel(x, shared_weights):
    idx_flat = x.reshape(TOK)
    run = functools.partial(
        pl.kernel,
        mesh=plsc.VectorSubcoreMesh(core_axis_name="c", subcore_axis_name="s"),
        out_type=jax.ShapeDtypeStruct((TOK, H), jnp.float32),
        scratch_types=[
            pltpu.VMEM((GROUPS_PER_W * G,), jnp.int32),
            pltpu.VMEM((G, H), jnp.float32),
            pltpu.SemaphoreType.DMA,
        ],
        compiler_params=pltpu.CompilerParams(use_tc_tiling_on_sc=False),
    )(_body)
    out = run(shared_weights, idx_flat)
    return out.reshape(B, L, H)


# speedup vs baseline: 1.1380x; 1.1380x over previous
import functools

import jax
import jax.numpy as jnp
from jax import lax
from jax.experimental import pallas as pl
from jax.experimental.pallas import tpu as pltpu
from jax.experimental.pallas import tpu_sc as plsc

VOCAB_SIZE = 1000000
H = 64
B = 1024
L = 200
TOK = B * L
G = 128
SCALE = float(H) ** 0.5

_info = plsc.get_sparse_core_info()
NC = _info.num_cores
NS = _info.num_subcores
NW = NC * NS
TOK_PER_W = TOK // NW
GROUPS_PER_W = TOK_PER_W // G
NBUF = 5
ROUNDS = GROUPS_PER_W // NBUF


def _body(table_hbm, idx_hbm, out_hbm, idx_v, rows_v, gsem, osem):
    wid = lax.axis_index("s") * NC + lax.axis_index("c")
    tok0 = wid * TOK_PER_W

    pltpu.sync_copy(idx_hbm.at[pl.ds(tok0, TOK_PER_W)], idx_v)

    def gather_of(g, b):
        return pltpu.make_async_copy(
            table_hbm.at[idx_v.at[pl.ds(g * G, G)]], rows_v.at[b], gsem.at[b]
        )

    def write_of(g, b):
        return pltpu.make_async_copy(
            rows_v.at[b], out_hbm.at[pl.ds(tok0 + g * G, G)], osem.at[b]
        )

    for b in range(NBUF):
        gather_of(b, b).start()

    def round_(o, carry):
        for b in range(NBUF):
            g = o * NBUF + b
            gather_of(g, b).wait()
            write_of(g, b).start()
        for b in range(NBUF):
            g = o * NBUF + b
            write_of(g, b).wait()

            @pl.when(o < ROUNDS - 1)
            def _():
                gather_of(g + NBUF, b).start()

        return carry

    lax.fori_loop(0, ROUNDS, round_, 0)


def _gather(table, idx_flat):
    run = functools.partial(
        pl.kernel,
        mesh=plsc.VectorSubcoreMesh(core_axis_name="c", subcore_axis_name="s"),
        out_type=jax.ShapeDtypeStruct((TOK, H), jnp.float32),
        scratch_types=[
            pltpu.VMEM((TOK_PER_W,), jnp.int32),
            pltpu.VMEM((NBUF, G, H), jnp.float32),
            pltpu.SemaphoreType.DMA((NBUF,)),
            pltpu.SemaphoreType.DMA((NBUF,)),
        ],
        compiler_params=pltpu.CompilerParams(use_tc_tiling_on_sc=False),
    )(_body)
    return run(table, idx_flat)


@jax.jit
def kernel(x, shared_weights):
    idx_flat = x.T.reshape(TOK)
    raw = _gather(shared_weights, idx_flat)
    raw = raw.reshape(L, B, H).transpose(1, 0, 2)
    scale = jnp.where(x == 0, jnp.float32(0.0), jnp.float32(SCALE))
    return raw * scale[..., None]


# speedup vs baseline: 1.1397x; 1.0015x over previous
import functools

import jax
import jax.numpy as jnp
from jax import lax
from jax.experimental import pallas as pl
from jax.experimental.pallas import tpu as pltpu
from jax.experimental.pallas import tpu_sc as plsc

VOCAB_SIZE = 1000000
H = 64
B = 1024
L = 200
TOK = B * L
G = 128
SCALE = float(H) ** 0.5

_info = plsc.get_sparse_core_info()
NC = _info.num_cores
NS = _info.num_subcores
NW = NC * NS
TOK_PER_W = TOK // NW
GROUPS_PER_W = TOK_PER_W // G
NBUF = 5
ROUNDS = GROUPS_PER_W // NBUF


def _body(table_hbm, idx_hbm, out_hbm, idx_v, rows_v, gsem, osem, isem):
    wid = lax.axis_index("s") * NC + lax.axis_index("c")
    tok0 = wid * TOK_PER_W

    def idx_dma(i):
        q = wid * GROUPS_PER_W + i
        return pltpu.make_async_copy(
            idx_hbm.at[q // 8, pl.ds((q % 8) * G, G)],
            idx_v.at[pl.ds(i * G, G)],
            isem,
        )

    def start_idx(i, c):
        idx_dma(i).start()
        return c

    def wait_idx(i, c):
        idx_dma(i).wait()
        return c

    lax.fori_loop(0, GROUPS_PER_W, start_idx, 0)
    lax.fori_loop(0, GROUPS_PER_W, wait_idx, 0)

    def gather_of(g, b):
        return pltpu.make_async_copy(
            table_hbm.at[idx_v.at[pl.ds(g * G, G)]], rows_v.at[b], gsem.at[b]
        )

    def write_of(g, b):
        return pltpu.make_async_copy(
            rows_v.at[b], out_hbm.at[pl.ds(tok0 + g * G, G)], osem.at[b]
        )

    for b in range(NBUF):
        gather_of(b, b).start()

    def round_(o, carry):
        for b in range(NBUF):
            g = o * NBUF + b
            gather_of(g, b).wait()
            write_of(g, b).start()
        for b in range(NBUF):
            g = o * NBUF + b
            write_of(g, b).wait()

            @pl.when(o < ROUNDS - 1)
            def _():
                gather_of(g + NBUF, b).start()

        return carry

    lax.fori_loop(0, ROUNDS, round_, 0)


def _gather(table, idx_flat):
    run = functools.partial(
        pl.kernel,
        mesh=plsc.VectorSubcoreMesh(core_axis_name="c", subcore_axis_name="s"),
        out_type=jax.ShapeDtypeStruct((TOK, H), jnp.float32),
        scratch_types=[
            pltpu.VMEM((TOK_PER_W,), jnp.int32),
            pltpu.VMEM((NBUF, G, H), jnp.float32),
            pltpu.SemaphoreType.DMA((NBUF,)),
            pltpu.SemaphoreType.DMA((NBUF,)),
            pltpu.SemaphoreType.DMA,
        ],
        compiler_params=pltpu.CompilerParams(use_tc_tiling_on_sc=False),
    )(_body)
    return run(table, idx_flat)


@jax.jit
def kernel(x, shared_weights):
    raw = _gather(shared_weights, x.T)
    raw = raw.reshape(L, B, H).transpose(1, 0, 2)
    scale = jnp.where(x == 0, jnp.float32(0.0), jnp.float32(SCALE))
    return raw * scale[..., None]
